# baseline (speedup 1.0000x reference)
import functools

import jax
import jax.numpy as jnp
from jax import lax
from jax.experimental import pallas as pl
from jax.experimental.pallas import tpu as pltpu
from jax.experimental.pallas import tpu_sc as plsc

_NUM_TYPES = 128
_N = 100000
_L = 16
_G = _N // _L
_NC = 2
_NS = 16
_NW = _NC * _NS
_GPW = _G // _NW
_EXTRA = _G - _GPW * _NW
_CG = 16
_CH_ELEMS = _CG * _L * _NUM_TYPES
_FG = 4
_F_ELEMS = _FG * _L * _NUM_TYPES
_NB = 3
_NCHUNK = 13
_TYPES_BUF = (_GPW + 1) * _L


@functools.partial(
    pl.kernel,
    out_type=jax.ShapeDtypeStruct((_N * _NUM_TYPES,), jnp.float32),
    mesh=plsc.VectorSubcoreMesh(core_axis_name="c", subcore_axis_name="s"),
    scratch_types=[
        pltpu.VMEM((_TYPES_BUF,), jnp.int32),
        pltpu.VMEM((_CH_ELEMS,), jnp.float32),
        pltpu.VMEM((_CH_ELEMS,), jnp.float32),
        pltpu.VMEM((_CH_ELEMS,), jnp.float32),
        pltpu.SemaphoreType.DMA,
        pltpu.SemaphoreType.DMA,
        pltpu.SemaphoreType.DMA,
        pltpu.SemaphoreType.DMA,
    ],
    compiler_params=pltpu.CompilerParams(needs_layout_passes=False),
)
def _onehot_sc(types_hbm, out_hbm, t_v, buf_a, buf_b, buf_c,
               sem_a, sem_b, sem_c, sem_t):
    cid = lax.axis_index("c")
    sid = lax.axis_index("s")
    wid = (sid * _NC + cid).astype(jnp.int32)
    g0 = wid * _GPW + jnp.minimum(wid, _EXTRA)
    gc = _GPW + (wid < _EXTRA).astype(jnp.int32)

    off_adj = jnp.where(g0 * _L + _TYPES_BUF > _N, _L, 0).astype(jnp.int32)
    types_cp = pltpu.async_copy(
        types_hbm.at[pl.ds(g0 * _L - off_adj, _TYPES_BUF)], t_v, sem_t)

    zvec = jnp.zeros((_L,), jnp.float32)
    ones = jnp.ones((_L,), jnp.float32)
    lane_off = lax.iota(jnp.int32, _L) * _NUM_TYPES

    def zero_buf(buf, lo, hi):
        def body(i, _):
            base = i * (_L * 16)
            for k in range(16):
                buf[pl.ds(base + k * _L, _L)] = zvec
            return 0
        lax.fori_loop(lo // (_L * 16), hi // (_L * 16), body, 0)

    def scatter_chunk(buf, cs, val, ng=_CG, unroll=4):
        def body(i, _):
            for k in range(unroll):
                g = i * unroll + k
                tv = t_v[pl.ds(off_adj + (cs + g) * _L, _L)]
                plsc.store_scatter(
                    buf, [lane_off + g * (_L * _NUM_TYPES) + tv], val)
            return 0
        lax.fori_loop(0, ng // unroll, body, 0)

    def chunk_start(c):
        return jnp.minimum(_FG + (c - 1) * _CG, gc - _CG)

    def out_at(rel_group, elems=_CH_ELEMS):
        return out_hbm.at[pl.ds((g0 + rel_group) * _L * _NUM_TYPES, elems)]

    bufs = (buf_a, buf_b, buf_c)
    sems = (sem_a, sem_b, sem_c)

    zero_buf(buf_a, 0, _F_ELEMS)
    types_cp.wait()
    scatter_chunk(buf_a, 0, ones, ng=_FG)
    pltpu.async_copy(buf_a.at[pl.ds(0, _F_ELEMS)], out_at(0, _F_ELEMS),
                     sem_a)
    for c in (1, 2):
        zero_buf(bufs[c], 0, _CH_ELEMS)
        scatter_chunk(bufs[c], _FG + (c - 1) * _CG, ones)
        pltpu.async_copy(bufs[c], out_at(_FG + (c - 1) * _CG), sems[c])
    zero_buf(buf_a, _F_ELEMS, _CH_ELEMS)

    pltpu.make_async_copy(buf_a.at[pl.ds(0, _F_ELEMS)], out_at(0, _F_ELEMS),
                          sem_a).wait()
    scatter_chunk(buf_a, 0, zvec, ng=_FG)
    scatter_chunk(buf_a, _FG + 2 * _CG, ones)
    pltpu.async_copy(buf_a, out_at(_FG + 2 * _CG), sem_a)

    def round_body(r, _):
        for b3 in range(_NB):
            c = 4 + r * _NB + b3
            b = bufs[(b3 + 1) % _NB]
            sem = sems[(b3 + 1) % _NB]
            prev = _FG + (c - 4) * _CG
            pltpu.make_async_copy(b, out_at(prev), sem).wait()
            scatter_chunk(b, prev, zvec)
            scatter_chunk(b, chunk_start(c), ones)
            pltpu.async_copy(b, out_at(chunk_start(c)), sem)
        return 0

    lax.fori_loop(0, (_NCHUNK - 4) // _NB, round_body, 0)

    pltpu.make_async_copy(buf_b, out_at(chunk_start(10)), sem_b).wait()
    pltpu.make_async_copy(buf_c, out_at(chunk_start(11)), sem_c).wait()
    pltpu.make_async_copy(buf_a, out_at(chunk_start(12)), sem_a).wait()


def kernel(pos, atom_types):
    del pos
    flat = _onehot_sc(atom_types.reshape(-1))
    return flat.reshape(_N, _NUM_TYPES)


# speedup vs baseline: 1.0132x; 1.0132x over previous
import functools

import jax
import jax.numpy as jnp
from jax import lax
from jax.experimental import pallas as pl
from jax.experimental.pallas import tpu as pltpu
from jax.experimental.pallas import tpu_sc as plsc

_NUM_TYPES = 128
_N = 100000
_L = 16
_G = _N // _L
_NC = 2
_NS = 16
_NW = _NC * _NS
_GPW = _G // _NW
_EXTRA = _G - _GPW * _NW
_CG = 16
_CH_ROWS = _CG * _L
_CH_ELEMS = _CH_ROWS * _NUM_TYPES
_NB = 3
_NFULL = _GPW // _CG
_NROUND = _NFULL // _NB
_TG = _GPW + 1 - _NFULL * _CG
_TAIL_ELEMS = _TG * _L * _NUM_TYPES
_TYPES_BUF = (_GPW + 1) * _L


@functools.partial(
    pl.kernel,
    out_type=jax.ShapeDtypeStruct((_N * _NUM_TYPES,), jnp.float32),
    mesh=plsc.VectorSubcoreMesh(core_axis_name="c", subcore_axis_name="s"),
    scratch_types=[
        pltpu.VMEM((_TYPES_BUF,), jnp.int32),
        pltpu.VMEM((_CH_ELEMS,), jnp.float32),
        pltpu.VMEM((_CH_ELEMS,), jnp.float32),
        pltpu.VMEM((_CH_ELEMS,), jnp.float32),
        pltpu.SemaphoreType.DMA,
        pltpu.SemaphoreType.DMA,
        pltpu.SemaphoreType.DMA,
        pltpu.SemaphoreType.DMA,
    ],
    compiler_params=pltpu.CompilerParams(needs_layout_passes=False),
)
def _onehot_sc(types_hbm, out_hbm, t_v, buf_a, buf_b, buf_c,
               sem_a, sem_b, sem_c, sem_t):
    cid = lax.axis_index("c")
    sid = lax.axis_index("s")
    wid = (sid * _NC + cid).astype(jnp.int32)
    g0 = wid * _GPW + jnp.minimum(wid, _EXTRA)
    gc = _GPW + (wid < _EXTRA).astype(jnp.int32)

    off_adj = jnp.where(g0 * _L + _TYPES_BUF > _N, _L, 0).astype(jnp.int32)
    types_cp = pltpu.async_copy(
        types_hbm.at[pl.ds(g0 * _L - off_adj, _TYPES_BUF)], t_v, sem_t)

    zvec = jnp.zeros((_L,), jnp.float32)
    ones = jnp.ones((_L,), jnp.float32)
    lane_off = lax.iota(jnp.int32, _L) * _NUM_TYPES

    def zero_buf(buf):
        def body(i, _):
            base = i * (_L * 16)
            for k in range(16):
                buf[pl.ds(base + k * _L, _L)] = zvec
            return 0
        lax.fori_loop(0, _CH_ELEMS // (_L * 16), body, 0)

    def scatter_chunk(buf, cs, val, ng=_CG, unroll=4):
        def body(i, _):
            for k in range(unroll):
                g = i * unroll + k
                tv = t_v[pl.ds(off_adj + (cs + g) * _L, _L)]
                plsc.store_scatter(
                    buf, [lane_off + g * (_L * _NUM_TYPES) + tv], val)
            return 0
        lax.fori_loop(0, ng // unroll, body, 0)

    def out_at(rel_group, elems=_CH_ELEMS):
        return out_hbm.at[pl.ds((g0 + rel_group) * _L * _NUM_TYPES, elems)]

    bufs = (buf_a, buf_b, buf_c)
    sems = (sem_a, sem_b, sem_c)

    types_waited = False
    for b in range(_NB):
        zero_buf(bufs[b])
        if not types_waited:
            types_cp.wait()
            types_waited = True
        scatter_chunk(bufs[b], b * _CG, ones)
        pltpu.async_copy(bufs[b], out_at(b * _CG), sems[b])

    def round_body(r, _):
        for b in range(_NB):
            c = r * _NB + b
            pltpu.make_async_copy(bufs[b], out_at((c - _NB) * _CG),
                                  sems[b]).wait()
            scatter_chunk(bufs[b], (c - _NB) * _CG, zvec)
            scatter_chunk(bufs[b], c * _CG, ones)
            pltpu.async_copy(bufs[b], out_at(c * _CG), sems[b])
        return 0

    lax.fori_loop(1, _NROUND, round_body, 0)

    ts = gc - _TG
    pltpu.make_async_copy(bufs[0], out_at((_NFULL - _NB) * _CG),
                          sems[0]).wait()
    scatter_chunk(bufs[0], (_NFULL - _NB) * _CG, zvec)
    scatter_chunk(bufs[0], ts, ones, ng=_TG)
    tail_cp = pltpu.async_copy(
        bufs[0].at[pl.ds(0, _TAIL_ELEMS)],
        out_hbm.at[pl.ds((g0 + ts) * _L * _NUM_TYPES, _TAIL_ELEMS)],
        sems[0])
    pltpu.make_async_copy(bufs[1], out_at((_NFULL - 2) * _CG), sems[1]).wait()
    pltpu.make_async_copy(bufs[2], out_at((_NFULL - 1) * _CG), sems[2]).wait()
    tail_cp.wait()


def kernel(pos, atom_types):
    del pos
    flat = _onehot_sc(atom_types.reshape(-1))
    return flat.reshape(_N, _NUM_TYPES)
